# baseline (speedup 1.0000x reference)
import functools

import jax
import jax.numpy as jnp
from jax.experimental import pallas as pl
from jax.experimental.pallas import tpu as pltpu

_LANE = 128
_SUBLANE = 8
_VMEM_LIMIT_BYTES = 64 * 1024 * 1024


def _round_up(a, m):
    return ((a + m - 1) // m) * m


def _head_kernel(xa_ref, xb_ref, xc_ref, wf_ref, bf_ref, wc_ref, bc_ref,
                 o_ref, *, inv_hw):
    total = (jnp.sum(xa_ref[...], axis=0) + jnp.sum(xb_ref[...], axis=0)
             + jnp.sum(xc_ref[...], axis=0))
    pooled = (total * inv_hw).astype(jnp.bfloat16)
    feat = jnp.dot(pooled, wf_ref[...].astype(jnp.bfloat16),
                   preferred_element_type=jnp.float32)
    feat = jnp.maximum(feat + bf_ref[...], 0.0).astype(jnp.bfloat16)
    logits = jax.lax.dot_general(
        feat, wc_ref[...].astype(jnp.bfloat16),
        (((1,), (1,)), ((), ())), preferred_element_type=jnp.float32)
    o_ref[...] = logits + bc_ref[...]


def _single_stream_kernel(x_ref, wf_ref, bf_ref, wc_ref, bc_ref, o_ref, *,
                          inv_hw):
    pooled = (jnp.sum(x_ref[...], axis=0) * inv_hw).astype(jnp.bfloat16)
    feat = jnp.dot(pooled, wf_ref[...].astype(jnp.bfloat16),
                   preferred_element_type=jnp.float32)
    feat = jnp.maximum(feat + bf_ref[...], 0.0).astype(jnp.bfloat16)
    logits = jax.lax.dot_general(
        feat, wc_ref[...].astype(jnp.bfloat16),
        (((1,), (1,)), ((), ())), preferred_element_type=jnp.float32)
    o_ref[...] = logits + bc_ref[...]


def kernel(x, w_feat, b_feat, w_cls, b_cls):
    n, c, h, w = x.shape
    hw = h * w
    f = w_feat.shape[1]
    k = w_cls.shape[1]

    tn = min(32, _round_up(n, _SUBLANE))
    n_pad = _round_up(n, tn)

    xs = jnp.transpose(x, (2, 3, 0, 1)).reshape(hw, n, c)
    if n_pad > n:
        xs = jnp.pad(xs, ((0, 0), (0, n_pad - n), (0, 0)))

    cost = pl.CostEstimate(
        flops=2 * n_pad * c * f + 2 * n_pad * f * k,
        transcendentals=0,
        bytes_accessed=4 * (xs.size + w_feat.size + w_cls.size + n_pad * k),
    )

    if hw % 7 == 0:
        u = hw // 7
        sa, ia, sb, ib, sc, ic = 3 * u, 0, 3 * u, 1, u, 6
    elif hw >= 3:
        sa, ia, sb, ib, sc, ic = hw - 2, 0, 1, hw - 2, 1, hw - 1
    else:
        sa = sb = sc = 0

    if sa:
        out = pl.pallas_call(
            functools.partial(_head_kernel, inv_hw=1.0 / float(hw)),
            out_shape=jax.ShapeDtypeStruct((n_pad, k), jnp.float32),
            grid=(n_pad // tn,),
            in_specs=[
                pl.BlockSpec((sa, tn, c), lambda i: (ia, i, 0)),
                pl.BlockSpec((sb, tn, c), lambda i: (ib, i, 0)),
                pl.BlockSpec((sc, tn, c), lambda i: (ic, i, 0)),
                pl.BlockSpec((c, f), lambda i: (0, 0)),
                pl.BlockSpec((1, f), lambda i: (0, 0)),
                pl.BlockSpec((k, f), lambda i: (0, 0)),
                pl.BlockSpec((1, k), lambda i: (0, 0)),
            ],
            out_specs=pl.BlockSpec((tn, k), lambda i: (i, 0)),
            compiler_params=pltpu.CompilerParams(
                dimension_semantics=("parallel",),
                vmem_limit_bytes=_VMEM_LIMIT_BYTES,
            ),
            cost_estimate=cost,
        )(xs, xs, xs, w_feat, b_feat, jnp.transpose(w_cls), b_cls)
    else:
        out = pl.pallas_call(
            functools.partial(_single_stream_kernel, inv_hw=1.0 / float(hw)),
            out_shape=jax.ShapeDtypeStruct((n_pad, k), jnp.float32),
            grid=(n_pad // tn,),
            in_specs=[
                pl.BlockSpec((hw, tn, c), lambda i: (0, i, 0)),
                pl.BlockSpec((c, f), lambda i: (0, 0)),
                pl.BlockSpec((1, f), lambda i: (0, 0)),
                pl.BlockSpec((k, f), lambda i: (0, 0)),
                pl.BlockSpec((1, k), lambda i: (0, 0)),
            ],
            out_specs=pl.BlockSpec((tn, k), lambda i: (i, 0)),
            compiler_params=pltpu.CompilerParams(
                dimension_semantics=("parallel",),
                vmem_limit_bytes=_VMEM_LIMIT_BYTES,
            ),
            cost_estimate=cost,
        )(xs, w_feat, b_feat, jnp.transpose(w_cls), b_cls)
    if n_pad > n:
        out = out[:n]
    return {"logits": out}


# speedup vs baseline: 1.0189x; 1.0189x over previous
import functools

import jax
import jax.numpy as jnp
from jax.experimental import pallas as pl
from jax.experimental.pallas import tpu as pltpu

_LANE = 128
_SUBLANE = 8
_VMEM_LIMIT_BYTES = 64 * 1024 * 1024


def _round_up(a, m):
    return ((a + m - 1) // m) * m


def _head_kernel(x_ref, wf_ref, bf_ref, wc_ref, bc_ref, o_ref, *, inv_hw):
    pooled = (jnp.sum(x_ref[...], axis=0) * inv_hw).astype(jnp.bfloat16)
    feat = jnp.dot(pooled, wf_ref[...].astype(jnp.bfloat16),
                   preferred_element_type=jnp.float32)
    feat = jnp.maximum(feat + bf_ref[...], 0.0).astype(jnp.bfloat16)
    logits = jax.lax.dot_general(
        feat, wc_ref[...].astype(jnp.bfloat16),
        (((1,), (1,)), ((), ())), preferred_element_type=jnp.float32)
    o_ref[...] = logits + bc_ref[...]


def kernel(x, w_feat, b_feat, w_cls, b_cls):
    n, c, h, w = x.shape
    hw = h * w
    f = w_feat.shape[1]
    k = w_cls.shape[1]

    tn = min(32, _round_up(n, _SUBLANE))
    n_pad = _round_up(n, tn)

    xs = jnp.transpose(x, (2, 3, 0, 1)).reshape(hw, n, c)
    if n_pad > n:
        xs = jnp.pad(xs, ((0, 0), (0, n_pad - n), (0, 0)))

    cost = pl.CostEstimate(
        flops=2 * n_pad * c * f + 2 * n_pad * f * k,
        transcendentals=0,
        bytes_accessed=4 * (xs.size + w_feat.size + w_cls.size + n_pad * k),
    )

    out = pl.pallas_call(
        functools.partial(_head_kernel, inv_hw=1.0 / float(hw)),
        out_shape=jax.ShapeDtypeStruct((n_pad, k), jnp.float32),
        grid=(n_pad // tn,),
        in_specs=[
            pl.BlockSpec((hw, tn, c), lambda i: (0, i, 0)),
            pl.BlockSpec((c, f), lambda i: (0, 0)),
            pl.BlockSpec((1, f), lambda i: (0, 0)),
            pl.BlockSpec((k, f), lambda i: (0, 0)),
            pl.BlockSpec((1, k), lambda i: (0, 0)),
        ],
        out_specs=pl.BlockSpec((tn, k), lambda i: (i, 0)),
        compiler_params=pltpu.CompilerParams(
            dimension_semantics=("parallel",),
            vmem_limit_bytes=_VMEM_LIMIT_BYTES,
        ),
        cost_estimate=cost,
    )(xs, w_feat, b_feat, jnp.transpose(w_cls), b_cls)
    if n_pad > n:
        out = out[:n]
    return {"logits": out}
